# initial kernel scaffold (unmeasured)
import jax
import jax.numpy as jnp
from jax import lax
from jax.experimental import pallas as pl
from jax.experimental.pallas import tpu as pltpu


def kernel(
    x,
):
    def body(*refs):
        pass

    out_shape = jax.ShapeDtypeStruct(..., jnp.float32)
    return pl.pallas_call(body, out_shape=out_shape)(...)


# baseline (device time: 152973 ns/iter reference)
import jax
import jax.numpy as jnp
from jax import lax
from jax.experimental import pallas as pl
from jax.experimental.pallas import tpu as pltpu

N_DEV = 4


def kernel(x):
    m_per, n = x.shape

    def body(x_ref, out_ref, comm_ref, send_sems, recv_sems):
        my_x = lax.axis_index("x")
        my_y = lax.axis_index("y")
        my_z = lax.axis_index("z")
        left = (my_y - 1) % N_DEV
        right = (my_y + 1) % N_DEV

        barrier_sem = pltpu.get_barrier_semaphore()
        for nbr in [left, right]:
            pl.semaphore_signal(
                barrier_sem, inc=1,
                device_id=(my_x, nbr, my_z),
                device_id_type=pl.DeviceIdType.MESH,
            )
        pl.semaphore_wait(barrier_sem, 2)

        out_ref[pl.ds(my_y * m_per, m_per), :] = x_ref[:, :]
        comm_ref[0, :, :] = x_ref[:, :]

        for h in range(N_DEV - 1):
            send_slot = h % 2
            recv_slot = (h + 1) % 2
            rdma = pltpu.make_async_remote_copy(
                src_ref=comm_ref.at[send_slot],
                dst_ref=comm_ref.at[recv_slot],
                send_sem=send_sems.at[send_slot],
                recv_sem=recv_sems.at[recv_slot],
                device_id=(my_x, right, my_z),
                device_id_type=pl.DeviceIdType.MESH,
            )
            rdma.start()
            rdma.wait()

            origin = (my_y - h - 1) % N_DEV
            out_ref[pl.ds(origin * m_per, m_per), :] = comm_ref[recv_slot, :, :]

    return pl.pallas_call(
        body,
        out_shape=jax.ShapeDtypeStruct((N_DEV * m_per, n), x.dtype),
        in_specs=[pl.BlockSpec(memory_space=pltpu.VMEM)],
        out_specs=pl.BlockSpec(memory_space=pltpu.VMEM),
        scratch_shapes=[
            pltpu.VMEM((2, m_per, n), x.dtype),
            pltpu.SemaphoreType.DMA((2,)),
            pltpu.SemaphoreType.DMA((2,)),
        ],
        compiler_params=pltpu.CompilerParams(collective_id=0),
    )(x)


# device time: 108611 ns/iter; 1.4084x vs baseline; 1.4084x over previous
import jax
import jax.numpy as jnp
from jax import lax
from jax.experimental import pallas as pl
from jax.experimental.pallas import tpu as pltpu

Y = 4
Q = 1


def kernel(x):
    m_per, n = x.shape
    half = m_per // 2
    sub = half // Q

    def body(x_ref, out_ref,
             send_m, send_p, send_xm, send_xp,
             recv_m, recv_p, recv_xm, recv_xp):
        my_x = lax.axis_index("x")
        my_y = lax.axis_index("y")
        my_z = lax.axis_index("z")
        has_left = my_y >= 1
        has_right = my_y <= Y - 2
        peer_d = (1 - my_x, my_y, my_z)
        left_d = (my_x, jnp.maximum(my_y - 1, 0), my_z)
        right_d = (my_x, jnp.minimum(my_y + 1, Y - 1), my_z)

        barrier = pltpu.get_barrier_semaphore()
        pl.semaphore_signal(barrier, inc=1, device_id=peer_d,
                            device_id_type=pl.DeviceIdType.MESH)

        @pl.when(has_left)
        def _():
            pl.semaphore_signal(barrier, inc=1, device_id=left_d,
                                device_id_type=pl.DeviceIdType.MESH)

        @pl.when(has_right)
        def _():
            pl.semaphore_signal(barrier, inc=1, device_id=right_d,
                                device_id_type=pl.DeviceIdType.MESH)

        interior = has_left & has_right

        @pl.when(interior)
        def _():
            pl.semaphore_wait(barrier, 3)

        @pl.when(~interior)
        def _():
            pl.semaphore_wait(barrier, 2)

        out_ref[pl.ds(my_y * m_per, m_per), :] = x_ref[:, :]

        def copy(c, h, q, dev, ssem, rsem):
            rows = pl.ds(c * m_per + h * half + q * sub, sub)
            return pltpu.make_async_remote_copy(
                src_ref=out_ref.at[rows, :],
                dst_ref=out_ref.at[rows, :],
                send_sem=ssem,
                recv_sem=rsem,
                device_id=dev,
                device_id_type=pl.DeviceIdType.MESH,
            )

        for s in range(Y - 1):
            cm = jnp.minimum(my_y + s, Y - 1)
            cp = jnp.maximum(my_y - s, 0)
            send_left_ok = has_left & (my_y + s <= Y - 1)
            send_right_ok = has_right & (my_y - s >= 0)
            recv_right_ok = my_y + 1 + s <= Y - 1
            recv_left_ok = my_y - 1 - s >= 0
            crm = jnp.minimum(my_y + 1 + s, Y - 1)
            crp = jnp.maximum(my_y - 1 - s, 0)

            for q in range(Q):
                @pl.when(send_left_ok)
                def _(s=s, q=q, cm=cm):
                    copy(cm, my_x, q, left_d,
                         send_m.at[s, q], recv_m.at[s, q]).start()

                @pl.when(send_right_ok)
                def _(s=s, q=q, cp=cp):
                    copy(cp, my_x, q, right_d,
                         send_p.at[s, q], recv_p.at[s, q]).start()

            for q in range(Q):
                @pl.when(recv_right_ok)
                def _(s=s, q=q, crm=crm):
                    copy(crm, my_x, q, right_d,
                         send_m.at[s, q], recv_m.at[s, q]).wait_recv()
                    copy(crm, my_x, q, peer_d,
                         send_xm.at[s, q], recv_xm.at[s, q]).start()

                @pl.when(recv_left_ok)
                def _(s=s, q=q, crp=crp):
                    copy(crp, my_x, q, left_d,
                         send_p.at[s, q], recv_p.at[s, q]).wait_recv()
                    copy(crp, my_x, q, peer_d,
                         send_xp.at[s, q], recv_xp.at[s, q]).start()

        oh = 1 - my_x
        for s in range(Y - 1):
            recv_right_ok = my_y + 1 + s <= Y - 1
            recv_left_ok = my_y - 1 - s >= 0
            send_left_ok = has_left & (my_y + s <= Y - 1)
            send_right_ok = has_right & (my_y - s >= 0)
            crm = jnp.minimum(my_y + 1 + s, Y - 1)
            crp = jnp.maximum(my_y - 1 - s, 0)
            cm = jnp.minimum(my_y + s, Y - 1)
            cp = jnp.maximum(my_y - s, 0)

            for q in range(Q):
                @pl.when(recv_right_ok)
                def _(s=s, q=q, crm=crm):
                    copy(crm, oh, q, peer_d,
                         send_xm.at[s, q], recv_xm.at[s, q]).wait_recv()
                    copy(crm, my_x, q, peer_d,
                         send_xm.at[s, q], recv_xm.at[s, q]).wait_send()

                @pl.when(recv_left_ok)
                def _(s=s, q=q, crp=crp):
                    copy(crp, oh, q, peer_d,
                         send_xp.at[s, q], recv_xp.at[s, q]).wait_recv()
                    copy(crp, my_x, q, peer_d,
                         send_xp.at[s, q], recv_xp.at[s, q]).wait_send()

                @pl.when(send_left_ok)
                def _(s=s, q=q, cm=cm):
                    copy(cm, my_x, q, left_d,
                         send_m.at[s, q], recv_m.at[s, q]).wait_send()

                @pl.when(send_right_ok)
                def _(s=s, q=q, cp=cp):
                    copy(cp, my_x, q, right_d,
                         send_p.at[s, q], recv_p.at[s, q]).wait_send()

    return pl.pallas_call(
        body,
        out_shape=jax.ShapeDtypeStruct((Y * m_per, n), x.dtype),
        in_specs=[pl.BlockSpec(memory_space=pltpu.VMEM)],
        out_specs=pl.BlockSpec(memory_space=pltpu.VMEM),
        scratch_shapes=[
            pltpu.SemaphoreType.DMA((Y - 1, Q)),
            pltpu.SemaphoreType.DMA((Y - 1, Q)),
            pltpu.SemaphoreType.DMA((Y - 1, Q)),
            pltpu.SemaphoreType.DMA((Y - 1, Q)),
            pltpu.SemaphoreType.DMA((Y - 1, Q)),
            pltpu.SemaphoreType.DMA((Y - 1, Q)),
            pltpu.SemaphoreType.DMA((Y - 1, Q)),
            pltpu.SemaphoreType.DMA((Y - 1, Q)),
        ],
        compiler_params=pltpu.CompilerParams(collective_id=0),
    )(x)


# device time: 92369 ns/iter; 1.6561x vs baseline; 1.1758x over previous
import jax
import jax.numpy as jnp
from jax import lax
from jax.experimental import pallas as pl
from jax.experimental.pallas import tpu as pltpu

Y = 4
Q = 4


def kernel(x):
    m_per, n = x.shape
    half = m_per // 2
    sub = half // Q

    def body(x_ref, out_ref,
             send_m, send_p, send_xm, send_xp,
             recv_m, recv_p, recv_xm, recv_xp):
        my_x = lax.axis_index("x")
        my_y = lax.axis_index("y")
        my_z = lax.axis_index("z")
        has_left = my_y >= 1
        has_right = my_y <= Y - 2
        peer_d = (1 - my_x, my_y, my_z)
        left_d = (my_x, jnp.maximum(my_y - 1, 0), my_z)
        right_d = (my_x, jnp.minimum(my_y + 1, Y - 1), my_z)

        barrier = pltpu.get_barrier_semaphore()
        pl.semaphore_signal(barrier, inc=1, device_id=peer_d,
                            device_id_type=pl.DeviceIdType.MESH)

        @pl.when(has_left)
        def _():
            pl.semaphore_signal(barrier, inc=1, device_id=left_d,
                                device_id_type=pl.DeviceIdType.MESH)

        @pl.when(has_right)
        def _():
            pl.semaphore_signal(barrier, inc=1, device_id=right_d,
                                device_id_type=pl.DeviceIdType.MESH)

        interior = has_left & has_right

        @pl.when(interior)
        def _():
            pl.semaphore_wait(barrier, 3)

        @pl.when(~interior)
        def _():
            pl.semaphore_wait(barrier, 2)

        out_ref[pl.ds(my_y * m_per, m_per), :] = x_ref[:, :]

        def copy(c, h, q, dev, ssem, rsem):
            rows = pl.ds(c * m_per + h * half + q * sub, sub)
            return pltpu.make_async_remote_copy(
                src_ref=out_ref.at[rows, :],
                dst_ref=out_ref.at[rows, :],
                send_sem=ssem,
                recv_sem=rsem,
                device_id=dev,
                device_id_type=pl.DeviceIdType.MESH,
            )

        for s in range(Y - 1):
            cm = jnp.minimum(my_y + s, Y - 1)
            cp = jnp.maximum(my_y - s, 0)
            send_left_ok = has_left & (my_y + s <= Y - 1)
            send_right_ok = has_right & (my_y - s >= 0)
            recv_right_ok = my_y + 1 + s <= Y - 1
            recv_left_ok = my_y - 1 - s >= 0
            crm = jnp.minimum(my_y + 1 + s, Y - 1)
            crp = jnp.maximum(my_y - 1 - s, 0)

            for q in range(Q):
                @pl.when(send_left_ok)
                def _(s=s, q=q, cm=cm):
                    copy(cm, my_x, q, left_d,
                         send_m.at[s, q], recv_m.at[s, q]).start()

                @pl.when(send_right_ok)
                def _(s=s, q=q, cp=cp):
                    copy(cp, my_x, q, right_d,
                         send_p.at[s, q], recv_p.at[s, q]).start()

            for q in range(Q):
                @pl.when(recv_right_ok)
                def _(s=s, q=q, crm=crm):
                    copy(crm, my_x, q, right_d,
                         send_m.at[s, q], recv_m.at[s, q]).wait_recv()
                    copy(crm, my_x, q, peer_d,
                         send_xm.at[s, q], recv_xm.at[s, q]).start()

                @pl.when(recv_left_ok)
                def _(s=s, q=q, crp=crp):
                    copy(crp, my_x, q, left_d,
                         send_p.at[s, q], recv_p.at[s, q]).wait_recv()
                    copy(crp, my_x, q, peer_d,
                         send_xp.at[s, q], recv_xp.at[s, q]).start()

        oh = 1 - my_x
        for s in range(Y - 1):
            recv_right_ok = my_y + 1 + s <= Y - 1
            recv_left_ok = my_y - 1 - s >= 0
            send_left_ok = has_left & (my_y + s <= Y - 1)
            send_right_ok = has_right & (my_y - s >= 0)
            crm = jnp.minimum(my_y + 1 + s, Y - 1)
            crp = jnp.maximum(my_y - 1 - s, 0)
            cm = jnp.minimum(my_y + s, Y - 1)
            cp = jnp.maximum(my_y - s, 0)

            for q in range(Q):
                @pl.when(recv_right_ok)
                def _(s=s, q=q, crm=crm):
                    copy(crm, oh, q, peer_d,
                         send_xm.at[s, q], recv_xm.at[s, q]).wait_recv()
                    copy(crm, my_x, q, peer_d,
                         send_xm.at[s, q], recv_xm.at[s, q]).wait_send()

                @pl.when(recv_left_ok)
                def _(s=s, q=q, crp=crp):
                    copy(crp, oh, q, peer_d,
                         send_xp.at[s, q], recv_xp.at[s, q]).wait_recv()
                    copy(crp, my_x, q, peer_d,
                         send_xp.at[s, q], recv_xp.at[s, q]).wait_send()

                @pl.when(send_left_ok)
                def _(s=s, q=q, cm=cm):
                    copy(cm, my_x, q, left_d,
                         send_m.at[s, q], recv_m.at[s, q]).wait_send()

                @pl.when(send_right_ok)
                def _(s=s, q=q, cp=cp):
                    copy(cp, my_x, q, right_d,
                         send_p.at[s, q], recv_p.at[s, q]).wait_send()

    return pl.pallas_call(
        body,
        out_shape=jax.ShapeDtypeStruct((Y * m_per, n), x.dtype),
        in_specs=[pl.BlockSpec(memory_space=pltpu.VMEM)],
        out_specs=pl.BlockSpec(memory_space=pltpu.VMEM),
        scratch_shapes=[
            pltpu.SemaphoreType.DMA((Y - 1, Q)),
            pltpu.SemaphoreType.DMA((Y - 1, Q)),
            pltpu.SemaphoreType.DMA((Y - 1, Q)),
            pltpu.SemaphoreType.DMA((Y - 1, Q)),
            pltpu.SemaphoreType.DMA((Y - 1, Q)),
            pltpu.SemaphoreType.DMA((Y - 1, Q)),
            pltpu.SemaphoreType.DMA((Y - 1, Q)),
            pltpu.SemaphoreType.DMA((Y - 1, Q)),
        ],
        compiler_params=pltpu.CompilerParams(collective_id=0),
    )(x)


# device time: 90174 ns/iter; 1.6964x vs baseline; 1.0243x over previous
import jax
import jax.numpy as jnp
from jax import lax
from jax.experimental import pallas as pl
from jax.experimental.pallas import tpu as pltpu

Y = 4
Q = 8


def kernel(x):
    m_per, n = x.shape
    half = m_per // 2
    sub = half // Q

    def body(x_ref, out_ref,
             send_m, send_p, send_xm, send_xp,
             recv_m, recv_p, recv_xm, recv_xp):
        my_x = lax.axis_index("x")
        my_y = lax.axis_index("y")
        my_z = lax.axis_index("z")
        has_left = my_y >= 1
        has_right = my_y <= Y - 2
        peer_d = (1 - my_x, my_y, my_z)
        left_d = (my_x, jnp.maximum(my_y - 1, 0), my_z)
        right_d = (my_x, jnp.minimum(my_y + 1, Y - 1), my_z)

        barrier = pltpu.get_barrier_semaphore()
        pl.semaphore_signal(barrier, inc=1, device_id=peer_d,
                            device_id_type=pl.DeviceIdType.MESH)

        @pl.when(has_left)
        def _():
            pl.semaphore_signal(barrier, inc=1, device_id=left_d,
                                device_id_type=pl.DeviceIdType.MESH)

        @pl.when(has_right)
        def _():
            pl.semaphore_signal(barrier, inc=1, device_id=right_d,
                                device_id_type=pl.DeviceIdType.MESH)

        interior = has_left & has_right

        @pl.when(interior)
        def _():
            pl.semaphore_wait(barrier, 3)

        @pl.when(~interior)
        def _():
            pl.semaphore_wait(barrier, 2)

        def copy(c, h, q, dev, ssem, rsem, from_input=False):
            rows = pl.ds(c * m_per + h * half + q * sub, sub)
            src = (x_ref.at[pl.ds(h * half + q * sub, sub), :]
                   if from_input else out_ref.at[rows, :])
            return pltpu.make_async_remote_copy(
                src_ref=src,
                dst_ref=out_ref.at[rows, :],
                send_sem=ssem,
                recv_sem=rsem,
                device_id=dev,
                device_id_type=pl.DeviceIdType.MESH,
            )

        for q in range(Q):
            @pl.when(has_left)
            def _(q=q):
                copy(my_y, my_x, q, left_d,
                     send_m.at[0, q], recv_m.at[0, q], from_input=True).start()

            @pl.when(has_right)
            def _(q=q):
                copy(my_y, my_x, q, right_d,
                     send_p.at[0, q], recv_p.at[0, q], from_input=True).start()

        out_ref[pl.ds(my_y * m_per, m_per), :] = x_ref[:, :]

        for s in range(Y - 1):
            cm = jnp.minimum(my_y + s, Y - 1)
            cp = jnp.maximum(my_y - s, 0)
            send_left_ok = has_left & (my_y + s <= Y - 1)
            send_right_ok = has_right & (my_y - s >= 0)
            recv_right_ok = my_y + 1 + s <= Y - 1
            recv_left_ok = my_y - 1 - s >= 0
            crm = jnp.minimum(my_y + 1 + s, Y - 1)
            crp = jnp.maximum(my_y - 1 - s, 0)

            if s > 0:
                for q in range(Q):
                    @pl.when(send_left_ok)
                    def _(s=s, q=q, cm=cm):
                        copy(cm, my_x, q, left_d,
                             send_m.at[s, q], recv_m.at[s, q]).start()

                    @pl.when(send_right_ok)
                    def _(s=s, q=q, cp=cp):
                        copy(cp, my_x, q, right_d,
                             send_p.at[s, q], recv_p.at[s, q]).start()

            for q in range(Q):
                @pl.when(recv_right_ok)
                def _(s=s, q=q, crm=crm):
                    copy(crm, my_x, q, right_d,
                         send_m.at[s, q], recv_m.at[s, q]).wait_recv()
                    copy(crm, my_x, q, peer_d,
                         send_xm.at[s, q], recv_xm.at[s, q]).start()

                @pl.when(recv_left_ok)
                def _(s=s, q=q, crp=crp):
                    copy(crp, my_x, q, left_d,
                         send_p.at[s, q], recv_p.at[s, q]).wait_recv()
                    copy(crp, my_x, q, peer_d,
                         send_xp.at[s, q], recv_xp.at[s, q]).start()

        oh = 1 - my_x
        for s in range(Y - 1):
            recv_right_ok = my_y + 1 + s <= Y - 1
            recv_left_ok = my_y - 1 - s >= 0
            send_left_ok = has_left & (my_y + s <= Y - 1)
            send_right_ok = has_right & (my_y - s >= 0)
            crm = jnp.minimum(my_y + 1 + s, Y - 1)
            crp = jnp.maximum(my_y - 1 - s, 0)
            cm = jnp.minimum(my_y + s, Y - 1)
            cp = jnp.maximum(my_y - s, 0)

            for q in range(Q):
                @pl.when(recv_right_ok)
                def _(s=s, q=q, crm=crm):
                    copy(crm, oh, q, peer_d,
                         send_xm.at[s, q], recv_xm.at[s, q]).wait_recv()
                    copy(crm, my_x, q, peer_d,
                         send_xm.at[s, q], recv_xm.at[s, q]).wait_send()

                @pl.when(recv_left_ok)
                def _(s=s, q=q, crp=crp):
                    copy(crp, oh, q, peer_d,
                         send_xp.at[s, q], recv_xp.at[s, q]).wait_recv()
                    copy(crp, my_x, q, peer_d,
                         send_xp.at[s, q], recv_xp.at[s, q]).wait_send()

                @pl.when(send_left_ok)
                def _(s=s, q=q, cm=cm):
                    copy(cm, my_x, q, left_d,
                         send_m.at[s, q], recv_m.at[s, q],
                         from_input=(s == 0)).wait_send()

                @pl.when(send_right_ok)
                def _(s=s, q=q, cp=cp):
                    copy(cp, my_x, q, right_d,
                         send_p.at[s, q], recv_p.at[s, q],
                         from_input=(s == 0)).wait_send()

    return pl.pallas_call(
        body,
        out_shape=jax.ShapeDtypeStruct((Y * m_per, n), x.dtype),
        in_specs=[pl.BlockSpec(memory_space=pltpu.VMEM)],
        out_specs=pl.BlockSpec(memory_space=pltpu.VMEM),
        scratch_shapes=[
            pltpu.SemaphoreType.DMA((Y - 1, Q)),
            pltpu.SemaphoreType.DMA((Y - 1, Q)),
            pltpu.SemaphoreType.DMA((Y - 1, Q)),
            pltpu.SemaphoreType.DMA((Y - 1, Q)),
            pltpu.SemaphoreType.DMA((Y - 1, Q)),
            pltpu.SemaphoreType.DMA((Y - 1, Q)),
            pltpu.SemaphoreType.DMA((Y - 1, Q)),
            pltpu.SemaphoreType.DMA((Y - 1, Q)),
        ],
        compiler_params=pltpu.CompilerParams(collective_id=0),
    )(x)


# device time: 90027 ns/iter; 1.6992x vs baseline; 1.0016x over previous
import jax
import jax.numpy as jnp
from jax import lax
from jax.experimental import pallas as pl
from jax.experimental.pallas import tpu as pltpu

Y = 4
Q = 8


def kernel(x):
    m_per, n = x.shape
    half = m_per // 2
    sub = half // Q

    def body(x_ref, out_ref,
             send_m, send_p, send_xm, send_xp,
             recv_m, recv_p, recv_xm, recv_xp, own_sem):
        my_x = lax.axis_index("x")
        my_y = lax.axis_index("y")
        my_z = lax.axis_index("z")
        has_left = my_y >= 1
        has_right = my_y <= Y - 2
        peer_d = (1 - my_x, my_y, my_z)
        left_d = (my_x, jnp.maximum(my_y - 1, 0), my_z)
        right_d = (my_x, jnp.minimum(my_y + 1, Y - 1), my_z)

        barrier = pltpu.get_barrier_semaphore()
        pl.semaphore_signal(barrier, inc=1, device_id=peer_d,
                            device_id_type=pl.DeviceIdType.MESH)

        @pl.when(has_left)
        def _():
            pl.semaphore_signal(barrier, inc=1, device_id=left_d,
                                device_id_type=pl.DeviceIdType.MESH)

        @pl.when(has_right)
        def _():
            pl.semaphore_signal(barrier, inc=1, device_id=right_d,
                                device_id_type=pl.DeviceIdType.MESH)

        interior = has_left & has_right

        @pl.when(interior)
        def _():
            pl.semaphore_wait(barrier, 3)

        @pl.when(~interior)
        def _():
            pl.semaphore_wait(barrier, 2)

        def copy(c, h, q, dev, ssem, rsem, from_input=False):
            rows = pl.ds(c * m_per + h * half + q * sub, sub)
            src = (x_ref.at[pl.ds(h * half + q * sub, sub), :]
                   if from_input else out_ref.at[rows, :])
            return pltpu.make_async_remote_copy(
                src_ref=src,
                dst_ref=out_ref.at[rows, :],
                send_sem=ssem,
                recv_sem=rsem,
                device_id=dev,
                device_id_type=pl.DeviceIdType.MESH,
            )

        for q in range(Q):
            @pl.when(has_left)
            def _(q=q):
                copy(my_y, my_x, q, left_d,
                     send_m.at[0, q], recv_m.at[0, q], from_input=True).start()

            @pl.when(has_right)
            def _(q=q):
                copy(my_y, my_x, q, right_d,
                     send_p.at[0, q], recv_p.at[0, q], from_input=True).start()

        own_copy = pltpu.make_async_copy(
            x_ref, out_ref.at[pl.ds(my_y * m_per, m_per), :], own_sem)
        own_copy.start()

        for s in range(Y - 1):
            cm = jnp.minimum(my_y + s, Y - 1)
            cp = jnp.maximum(my_y - s, 0)
            send_left_ok = has_left & (my_y + s <= Y - 1)
            send_right_ok = has_right & (my_y - s >= 0)
            recv_right_ok = my_y + 1 + s <= Y - 1
            recv_left_ok = my_y - 1 - s >= 0
            crm = jnp.minimum(my_y + 1 + s, Y - 1)
            crp = jnp.maximum(my_y - 1 - s, 0)

            if s > 0:
                for q in range(Q):
                    @pl.when(send_left_ok)
                    def _(s=s, q=q, cm=cm):
                        copy(cm, my_x, q, left_d,
                             send_m.at[s, q], recv_m.at[s, q]).start()

                    @pl.when(send_right_ok)
                    def _(s=s, q=q, cp=cp):
                        copy(cp, my_x, q, right_d,
                             send_p.at[s, q], recv_p.at[s, q]).start()

            for q in range(Q):
                @pl.when(recv_right_ok)
                def _(s=s, q=q, crm=crm):
                    copy(crm, my_x, q, right_d,
                         send_m.at[s, q], recv_m.at[s, q]).wait_recv()
                    copy(crm, my_x, q, peer_d,
                         send_xm.at[s, q], recv_xm.at[s, q]).start()

                @pl.when(recv_left_ok)
                def _(s=s, q=q, crp=crp):
                    copy(crp, my_x, q, left_d,
                         send_p.at[s, q], recv_p.at[s, q]).wait_recv()
                    copy(crp, my_x, q, peer_d,
                         send_xp.at[s, q], recv_xp.at[s, q]).start()

        oh = 1 - my_x
        for s in range(Y - 1):
            recv_right_ok = my_y + 1 + s <= Y - 1
            recv_left_ok = my_y - 1 - s >= 0
            send_left_ok = has_left & (my_y + s <= Y - 1)
            send_right_ok = has_right & (my_y - s >= 0)
            crm = jnp.minimum(my_y + 1 + s, Y - 1)
            crp = jnp.maximum(my_y - 1 - s, 0)
            cm = jnp.minimum(my_y + s, Y - 1)
            cp = jnp.maximum(my_y - s, 0)

            for q in range(Q):
                @pl.when(recv_right_ok)
                def _(s=s, q=q, crm=crm):
                    copy(crm, oh, q, peer_d,
                         send_xm.at[s, q], recv_xm.at[s, q]).wait_recv()
                    copy(crm, my_x, q, peer_d,
                         send_xm.at[s, q], recv_xm.at[s, q]).wait_send()

                @pl.when(recv_left_ok)
                def _(s=s, q=q, crp=crp):
                    copy(crp, oh, q, peer_d,
                         send_xp.at[s, q], recv_xp.at[s, q]).wait_recv()
                    copy(crp, my_x, q, peer_d,
                         send_xp.at[s, q], recv_xp.at[s, q]).wait_send()

                @pl.when(send_left_ok)
                def _(s=s, q=q, cm=cm):
                    copy(cm, my_x, q, left_d,
                         send_m.at[s, q], recv_m.at[s, q],
                         from_input=(s == 0)).wait_send()

                @pl.when(send_right_ok)
                def _(s=s, q=q, cp=cp):
                    copy(cp, my_x, q, right_d,
                         send_p.at[s, q], recv_p.at[s, q],
                         from_input=(s == 0)).wait_send()

        own_copy.wait()

    return pl.pallas_call(
        body,
        out_shape=jax.ShapeDtypeStruct((Y * m_per, n), x.dtype),
        in_specs=[pl.BlockSpec(memory_space=pltpu.VMEM)],
        out_specs=pl.BlockSpec(memory_space=pl.ANY),
        scratch_shapes=[
            pltpu.SemaphoreType.DMA((Y - 1, Q)),
            pltpu.SemaphoreType.DMA((Y - 1, Q)),
            pltpu.SemaphoreType.DMA((Y - 1, Q)),
            pltpu.SemaphoreType.DMA((Y - 1, Q)),
            pltpu.SemaphoreType.DMA((Y - 1, Q)),
            pltpu.SemaphoreType.DMA((Y - 1, Q)),
            pltpu.SemaphoreType.DMA((Y - 1, Q)),
            pltpu.SemaphoreType.DMA((Y - 1, Q)),
            pltpu.SemaphoreType.DMA,
        ],
        compiler_params=pltpu.CompilerParams(collective_id=0),
    )(x)
